# baseline (speedup 1.0000x reference)
import functools

import jax
import jax.numpy as jnp
from jax import lax
from jax.experimental import pallas as pl
from jax.experimental.pallas import tpu as pltpu
from jax.experimental.pallas import tpu_sc as plsc

N = 10000
D = 128
S = 32
NW = 32
NP = 10240
CHUNK = NP // NW
LANES = 16
NCH = D // LANES
MM_BLK = 1024


def _mm_body(x_ref, w_ref, o_ref):
    o_ref[...] = jnp.dot(x_ref[...], w_ref[...],
                         preferred_element_type=jnp.float32)


def _matmul(xp, W):
    return pl.pallas_call(
        _mm_body,
        grid=(NP // MM_BLK,),
        in_specs=[pl.BlockSpec((MM_BLK, D), lambda i: (i, 0)),
                  pl.BlockSpec((D, D), lambda i: (0, 0))],
        out_specs=pl.BlockSpec((MM_BLK, D), lambda i: (i, 0)),
        out_shape=jax.ShapeDtypeStruct((NP, D), jnp.float32),
    )(xp, W)


def _sc_gather_combine(y, adjp, b):
    mesh = plsc.VectorSubcoreMesh(core_axis_name="c", subcore_axis_name="s")

    @functools.partial(
        pl.kernel,
        mesh=mesh,
        out_type=jax.ShapeDtypeStruct((NP, D), jnp.float32),
        scratch_types=[
            pltpu.VMEM((CHUNK, S), jnp.int32),
            pltpu.VMEM((CHUNK, D), jnp.float32),
            pltpu.VMEM((CHUNK, D), jnp.float32),
            pltpu.VMEM((S, D), jnp.float32),
            pltpu.VMEM((D,), jnp.float32),
            pltpu.SemaphoreType.DMA,
        ],
    )
    def k(y_hbm, adj_hbm, b_hbm, out_hbm, adj_v, y_v, out_v, rows_v, b_v, sem):
        wid = lax.axis_index("s") * 2 + lax.axis_index("c")
        base = wid * CHUNK
        pltpu.sync_copy(adj_hbm.at[pl.ds(base, CHUNK)], adj_v)
        pltpu.sync_copy(y_hbm.at[pl.ds(base, CHUNK)], y_v)
        pltpu.sync_copy(b_hbm, b_v)

        def node(i, carry):
            pltpu.async_copy(y_hbm.at[adj_v.at[i]], rows_v, sem).wait()
            for c in range(NCH):
                sl = pl.ds(c * LANES, LANES)
                acc = rows_v[0, sl]
                for j in range(1, S):
                    acc = acc + rows_v[j, sl]
                out_v[i, sl] = 0.5 * y_v[i, sl] + (0.5 / S) * acc + b_v[sl]
            return carry

        lax.fori_loop(0, CHUNK, node, 0)
        pltpu.sync_copy(out_v, out_hbm.at[pl.ds(base, CHUNK)])

    return k(y, adjp, b)


def kernel(x, neighbor_adj, W, b):
    xp = jnp.pad(x, ((0, NP - N), (0, 0)))
    adjp = jnp.pad(neighbor_adj.astype(jnp.int32), ((0, NP - N), (0, 0)))
    y = _matmul(xp, W)
    outp = _sc_gather_combine(y, adjp, b)
    return outp[:N]


# speedup vs baseline: 1.5205x; 1.5205x over previous
import functools

import jax
import jax.numpy as jnp
from jax import lax
from jax.experimental import pallas as pl
from jax.experimental.pallas import tpu as pltpu
from jax.experimental.pallas import tpu_sc as plsc

N = 10000
D = 128
S = 32
NW = 32
NP = 10240
CHUNK = NP // NW
LANES = 16
NCH = D // LANES
BATCH = 4
BS = BATCH * S
NSTEPS = CHUNK // BATCH
MM_BLK = 1024


def _mm_body(x_ref, w_ref, o_ref):
    o_ref[...] = jnp.dot(x_ref[...], w_ref[...],
                         preferred_element_type=jnp.float32)


def _matmul(xp, W):
    return pl.pallas_call(
        _mm_body,
        grid=(NP // MM_BLK,),
        in_specs=[pl.BlockSpec((MM_BLK, D), lambda i: (i, 0)),
                  pl.BlockSpec((D, D), lambda i: (0, 0))],
        out_specs=pl.BlockSpec((MM_BLK, D), lambda i: (i, 0)),
        out_shape=jax.ShapeDtypeStruct((NP, D), jnp.float32),
    )(xp, W)


def _sc_gather_combine(y, adj_flat, b):
    mesh = plsc.VectorSubcoreMesh(core_axis_name="c", subcore_axis_name="s")

    @functools.partial(
        pl.kernel,
        mesh=mesh,
        out_type=jax.ShapeDtypeStruct((NP, D), jnp.float32),
        scratch_types=[
            pltpu.VMEM((CHUNK * S,), jnp.int32),
            pltpu.VMEM((CHUNK, D), jnp.float32),
            pltpu.VMEM((BS, D), jnp.float32),
            pltpu.VMEM((BS, D), jnp.float32),
            pltpu.VMEM((D,), jnp.float32),
            pltpu.SemaphoreType.DMA,
            pltpu.SemaphoreType.DMA,
        ],
    )
    def k(y_hbm, adj_hbm, b_hbm, out_hbm,
          adj_v, y_v, buf0, buf1, b_v, sem0, sem1):
        wid = lax.axis_index("s") * 2 + lax.axis_index("c")
        base = wid * CHUNK
        pltpu.sync_copy(adj_hbm.at[pl.ds(base * S, CHUNK * S)], adj_v)
        pltpu.sync_copy(y_hbm.at[pl.ds(base, CHUNK)], y_v)
        pltpu.sync_copy(b_hbm, b_v)

        def gather(s, buf, sem):
            idx = adj_v.at[pl.ds(s * BS, BS)]
            return pltpu.make_async_copy(y_hbm.at[idx], buf, sem)

        def accum(s, buf):
            for t in range(BATCH):
                i = s * BATCH + t
                row = t * S
                accs = [buf[row, pl.ds(c * LANES, LANES)] for c in range(NCH)]
                for j in range(1, S):
                    for c in range(NCH):
                        accs[c] = accs[c] + buf[row + j, pl.ds(c * LANES, LANES)]
                for c in range(NCH):
                    sl = pl.ds(c * LANES, LANES)
                    y_v[i, sl] = (0.5 * y_v[i, sl]
                                  + (0.5 / S) * accs[c] + b_v[sl])

        gather(0, buf0, sem0).start()

        def body2(g2, carry):
            s0 = 2 * g2
            s1 = s0 + 1
            gather(s1, buf1, sem1).start()
            gather(s0, buf0, sem0).wait()
            accum(s0, buf0)

            @pl.when(g2 < NSTEPS // 2 - 1)
            def _():
                gather(s1 + 1, buf0, sem0).start()

            gather(s1, buf1, sem1).wait()
            accum(s1, buf1)
            return carry

        lax.fori_loop(0, NSTEPS // 2, body2, 0)
        pltpu.sync_copy(y_v, out_hbm.at[pl.ds(base, CHUNK)])

    return k(y, adj_flat, b)


def kernel(x, neighbor_adj, W, b):
    xp = jnp.pad(x, ((0, NP - N), (0, 0)))
    adjp = jnp.pad(neighbor_adj.astype(jnp.int32), ((0, NP - N), (0, 0)))
    y = _matmul(xp, W)
    outp = _sc_gather_combine(y, adjp.reshape(NP * S), b)
    return outp[:N]


# speedup vs baseline: 2.6536x; 1.7452x over previous
import functools

import jax
import jax.numpy as jnp
from jax import lax
from jax.experimental import pallas as pl
from jax.experimental.pallas import tpu as pltpu
from jax.experimental.pallas import tpu_sc as plsc

N = 10000
D = 128
S = 32
NW = 32
NP = 10240
CHUNK = NP // NW
LANES = 16
NCH = D // LANES
NG = D // 32
DW = D // 2
BATCH = 4
BS = BATCH * S
NSTEPS = CHUNK // BATCH
NBUF = 4
MM_BLK = 1024


def _mm_body(x_ref, w_ref, o_ref):
    o_ref[...] = jnp.dot(x_ref[...], w_ref[...],
                         preferred_element_type=jnp.float32)


def _matmul(xp, W):
    return pl.pallas_call(
        _mm_body,
        grid=(NP // MM_BLK,),
        in_specs=[pl.BlockSpec((MM_BLK, D), lambda i: (i, 0)),
                  pl.BlockSpec((D, D), lambda i: (0, 0))],
        out_specs=pl.BlockSpec((MM_BLK, D), lambda i: (i, 0)),
        out_shape=jax.ShapeDtypeStruct((NP, D), jnp.float32),
    )(xp, W)


def _pack_bf16_pairs(y):
    yb = y.astype(jnp.bfloat16).reshape(NP, NG, 2, LANES)
    yb = yb.transpose(0, 1, 3, 2)
    return lax.bitcast_convert_type(yb, jnp.int32).reshape(NP, DW)


def _unpack2(v):
    lo = lax.bitcast_convert_type(v << 16, jnp.float32)
    hi = lax.bitcast_convert_type(v, jnp.float32)
    return lo, hi


def _sc_gather_combine(ybi, adj_flat, b):
    mesh = plsc.VectorSubcoreMesh(core_axis_name="c", subcore_axis_name="s")

    @functools.partial(
        pl.kernel,
        mesh=mesh,
        compiler_params=pltpu.CompilerParams(needs_layout_passes=False,
                                             use_tc_tiling_on_sc=False),
        out_type=jax.ShapeDtypeStruct((NP, D), jnp.float32),
        scratch_types=[
            pltpu.VMEM((CHUNK * S,), jnp.int32),
            pltpu.VMEM((CHUNK, DW), jnp.int32),
            pltpu.VMEM((CHUNK, D), jnp.float32),
            pltpu.VMEM((BS, DW), jnp.int32),
            pltpu.VMEM((BS, DW), jnp.int32),
            pltpu.VMEM((BS, DW), jnp.int32),
            pltpu.VMEM((BS, DW), jnp.int32),
            pltpu.VMEM((D,), jnp.float32),
            pltpu.SemaphoreType.DMA,
            pltpu.SemaphoreType.DMA,
            pltpu.SemaphoreType.DMA,
            pltpu.SemaphoreType.DMA,
        ],
    )
    def k(yb_hbm, adj_hbm, b_hbm, out_hbm,
          adj_v, my_v, out_v, r0, r1, r2, r3, b_v, s0, s1, s2, s3):
        bufs = (r0, r1, r2, r3)
        sems = (s0, s1, s2, s3)
        wid = lax.axis_index("s") * 2 + lax.axis_index("c")
        base = wid * CHUNK
        pltpu.sync_copy(adj_hbm.at[pl.ds(base * S, CHUNK * S)], adj_v)
        pltpu.sync_copy(yb_hbm.at[pl.ds(base, CHUNK)], my_v)
        pltpu.sync_copy(b_hbm, b_v)

        def gather(s, buf, sem):
            idx = adj_v.at[pl.ds(s * BS, BS)]
            return pltpu.make_async_copy(yb_hbm.at[idx], buf, sem)

        def accum(s, buf):
            for t in range(BATCH):
                i = s * BATCH + t
                row = t * S

                def rowloop(jj, accs, row=row, buf=buf):
                    accs = list(accs)
                    r = row + jj * 4
                    for u in range(4):
                        for g in range(NG):
                            v = buf[r + u, pl.ds(g * LANES, LANES)]
                            lo, hi = _unpack2(v)
                            accs[2 * g] = accs[2 * g] + lo
                            accs[2 * g + 1] = accs[2 * g + 1] + hi
                    return tuple(accs)

                zero = jnp.zeros((LANES,), jnp.float32)
                accs = lax.fori_loop(0, S // 4, rowloop, (zero,) * NCH)
                for g in range(NG):
                    v = my_v[i, pl.ds(g * LANES, LANES)]
                    lo, hi = _unpack2(v)
                    sl0 = pl.ds((2 * g) * LANES, LANES)
                    sl1 = pl.ds((2 * g + 1) * LANES, LANES)
                    out_v[i, sl0] = 0.5 * lo + (0.5 / S) * accs[2 * g] + b_v[sl0]
                    out_v[i, sl1] = 0.5 * hi + (0.5 / S) * accs[2 * g + 1] + b_v[sl1]

        for p in range(NBUF):
            gather(p, bufs[p], sems[p]).start()

        def body(g4, carry):
            for p in range(NBUF):
                s = NBUF * g4 + p
                gather(s, bufs[p], sems[p]).wait()
                accum(s, bufs[p])

                @pl.when(s < NSTEPS - NBUF)
                def _(s=s, p=p):
                    gather(s + NBUF, bufs[p], sems[p]).start()

            return carry

        lax.fori_loop(0, NSTEPS // NBUF, body, 0)
        pltpu.sync_copy(out_v, out_hbm.at[pl.ds(base, CHUNK)])

    return k(ybi, adj_flat, b)


def kernel(x, neighbor_adj, W, b):
    xp = jnp.pad(x, ((0, NP - N), (0, 0)))
    adjp = jnp.pad(neighbor_adj.astype(jnp.int32), ((0, NP - N), (0, 0)))
    y = _matmul(xp, W)
    ybi = _pack_bf16_pairs(y)
    outp = _sc_gather_combine(ybi, adjp.reshape(NP * S), b)
    return outp[:N]


# speedup vs baseline: 7.6525x; 2.8838x over previous
import functools

import jax
import jax.numpy as jnp
from jax import lax
from jax.experimental import pallas as pl
from jax.experimental.pallas import tpu as pltpu
from jax.experimental.pallas import tpu_sc as plsc

N = 10000
D = 128
S = 32
NW = 32
NP = 10240
CHUNK = NP // NW
LANES = 16
NCH = D // LANES
NG = D // 32
DW = D // 2
BATCH = 4
BS = BATCH * S
NSTEPS = CHUNK // BATCH
NBUF = 4
MM_BLK = 1024


def _mm_body(x_ref, w_ref, o_ref):
    o_ref[...] = jnp.dot(x_ref[...], w_ref[...],
                         preferred_element_type=jnp.float32)


def _matmul(xp, W):
    return pl.pallas_call(
        _mm_body,
        grid=(NP // MM_BLK,),
        in_specs=[pl.BlockSpec((MM_BLK, D), lambda i: (i, 0)),
                  pl.BlockSpec((D, D), lambda i: (0, 0))],
        out_specs=pl.BlockSpec((MM_BLK, D), lambda i: (i, 0)),
        out_shape=jax.ShapeDtypeStruct((NP, D), jnp.float32),
    )(xp, W)


def _pack_bf16_pairs(y):
    yb = y.astype(jnp.bfloat16).reshape(NP, NG, 2, LANES)
    yb = yb.transpose(0, 1, 3, 2)
    return lax.bitcast_convert_type(yb, jnp.int32).reshape(NP, DW)


def _unpack2(v):
    lo = lax.bitcast_convert_type(v << 16, jnp.float32)
    hi = lax.bitcast_convert_type(v, jnp.float32)
    return lo, hi


def _sc_gather_combine(ybi, adj_flat, b):
    mesh = plsc.VectorSubcoreMesh(core_axis_name="c", subcore_axis_name="s")

    @functools.partial(
        pl.kernel,
        mesh=mesh,
        compiler_params=pltpu.CompilerParams(needs_layout_passes=False,
                                             use_tc_tiling_on_sc=False),
        out_type=jax.ShapeDtypeStruct((NP, D), jnp.float32),
        scratch_types=[
            pltpu.VMEM_SHARED((NP, DW), jnp.int32),
            pltpu.VMEM((CHUNK * S,), jnp.int32),
            pltpu.VMEM((CHUNK, DW), jnp.int32),
            pltpu.VMEM((BATCH, D), jnp.float32),
            pltpu.VMEM((BATCH, D), jnp.float32),
            pltpu.VMEM((BS, DW), jnp.int32),
            pltpu.VMEM((BS, DW), jnp.int32),
            pltpu.VMEM((BS, DW), jnp.int32),
            pltpu.VMEM((BS, DW), jnp.int32),
            pltpu.VMEM((D,), jnp.float32),
            pltpu.SemaphoreType.DMA,
            pltpu.SemaphoreType.DMA,
            pltpu.SemaphoreType.DMA,
            pltpu.SemaphoreType.DMA,
            pltpu.SemaphoreType.DMA,
            pltpu.SemaphoreType.DMA,
        ],
    )
    def k(yb_hbm, adj_hbm, b_hbm, out_hbm,
          yb_sp, adj_v, my_v, ob0, ob1, r0, r1, r2, r3, b_v,
          s0, s1, s2, s3, os0, os1):
        bufs = (r0, r1, r2, r3)
        sems = (s0, s1, s2, s3)
        obufs = (ob0, ob1)
        osems = (os0, os1)
        sid = lax.axis_index("s")
        wid = sid * 2 + lax.axis_index("c")
        base = wid * CHUNK
        sl16 = pl.ds(sid * (NP // 16), NP // 16)
        pltpu.sync_copy(yb_hbm.at[sl16], yb_sp.at[sl16])
        pltpu.sync_copy(adj_hbm.at[pl.ds(base * S, CHUNK * S)], adj_v)
        pltpu.sync_copy(b_hbm, b_v)
        plsc.subcore_barrier()
        pltpu.sync_copy(yb_sp.at[pl.ds(base, CHUNK)], my_v)

        def gather(s, buf, sem):
            idx = adj_v.at[pl.ds(s * BS, BS)]
            return pltpu.make_async_copy(yb_sp.at[idx], buf, sem)

        def ocopy(s, q):
            dst = out_hbm.at[pl.ds(base + s * BATCH, BATCH)]
            return pltpu.make_async_copy(obufs[q], dst, osems[q])

        def accum(s, buf, ob):
            for t in range(BATCH):
                i = s * BATCH + t
                row = t * S

                def rowloop(jj, accs, row=row, buf=buf):
                    accs = list(accs)
                    r = row + jj * 4
                    for u in range(4):
                        for g in range(NG):
                            v = buf[r + u, pl.ds(g * LANES, LANES)]
                            lo, hi = _unpack2(v)
                            accs[2 * g] = accs[2 * g] + lo
                            accs[2 * g + 1] = accs[2 * g + 1] + hi
                    return tuple(accs)

                zero = jnp.zeros((LANES,), jnp.float32)
                accs = lax.fori_loop(0, S // 4, rowloop, (zero,) * NCH)
                for g in range(NG):
                    v = my_v[i, pl.ds(g * LANES, LANES)]
                    lo, hi = _unpack2(v)
                    sl0 = pl.ds((2 * g) * LANES, LANES)
                    sl1 = pl.ds((2 * g + 1) * LANES, LANES)
                    ob[t, sl0] = 0.5 * lo + (0.5 / S) * accs[2 * g] + b_v[sl0]
                    ob[t, sl1] = 0.5 * hi + (0.5 / S) * accs[2 * g + 1] + b_v[sl1]

        for p in range(NBUF):
            gather(p, bufs[p], sems[p]).start()

        def body(g4, carry):
            for p in range(NBUF):
                s = NBUF * g4 + p
                q = p % 2
                gather(s, bufs[p], sems[p]).wait()

                @pl.when(s >= 2)
                def _(s=s, q=q):
                    ocopy(s - 2, q).wait()

                accum(s, bufs[p], obufs[q])
                ocopy(s, q).start()

                @pl.when(s < NSTEPS - NBUF)
                def _(s=s, p=p):
                    gather(s + NBUF, bufs[p], sems[p]).start()

            return carry

        lax.fori_loop(0, NSTEPS // NBUF, body, 0)
        ocopy(NSTEPS - 2, 0).wait()
        ocopy(NSTEPS - 1, 1).wait()

    return k(ybi, adj_flat, b)


def kernel(x, neighbor_adj, W, b):
    xp = jnp.pad(x, ((0, NP - N), (0, 0)))
    adjp = jnp.pad(neighbor_adj.astype(jnp.int32), ((0, NP - N), (0, 0)))
    y = _matmul(xp, W)
    ybi = _pack_bf16_pairs(y)
    outp = _sc_gather_combine(ybi, adjp.reshape(NP * S), b)
    return outp[:N]


# speedup vs baseline: 9.8755x; 1.2905x over previous
import functools

import jax
import jax.numpy as jnp
from jax import lax
from jax.experimental import pallas as pl
from jax.experimental.pallas import tpu as pltpu
from jax.experimental.pallas import tpu_sc as plsc

N = 10000
D = 128
S = 32
NW = 32
CHUNK = 320
LANES = 16
NCH = D // LANES
NG = 4
DW = D // 2
BATCH = 4
BS = BATCH * S
NSTEPS = CHUNK // BATCH
NBUF = 4
QUARTER = CHUNK // 4
MM_BLK = 1000


def _mm_pack_body(x_ref, w_ref, o_ref):
    y = jnp.dot(x_ref[...], w_ref[...], preferred_element_type=jnp.float32)

    def rne16(i):
        return (i + 0x7FFF + ((i >> 16) & 1)) >> 16

    ilo = lax.bitcast_convert_type(y[:, :DW], jnp.int32)
    ihi = lax.bitcast_convert_type(y[:, DW:], jnp.int32)
    o_ref[...] = (rne16(ilo) & 0xFFFF) | (rne16(ihi) << 16)


def _matmul_pack(x, W):
    return pl.pallas_call(
        _mm_pack_body,
        grid=(N // MM_BLK,),
        in_specs=[pl.BlockSpec((MM_BLK, D), lambda i: (i, 0)),
                  pl.BlockSpec((D, D), lambda i: (0, 0))],
        out_specs=pl.BlockSpec((MM_BLK, DW), lambda i: (i, 0)),
        out_shape=jax.ShapeDtypeStruct((N, DW), jnp.int32),
    )(x, W)


def _unpack2(v):
    lo = lax.bitcast_convert_type(v << 16, jnp.float32)
    hi = lax.bitcast_convert_type(v, jnp.float32)
    return lo, hi


def _sc_gather_combine(ybi, adj_flat, b):
    mesh = plsc.VectorSubcoreMesh(core_axis_name="c", subcore_axis_name="s")

    @functools.partial(
        pl.kernel,
        mesh=mesh,
        compiler_params=pltpu.CompilerParams(needs_layout_passes=False,
                                             use_tc_tiling_on_sc=False),
        out_type=jax.ShapeDtypeStruct((N, D), jnp.float32),
        scratch_types=[
            pltpu.VMEM_SHARED((N, DW), jnp.int32),
            pltpu.VMEM((CHUNK * S,), jnp.int32),
            pltpu.VMEM((CHUNK, DW), jnp.int32),
            pltpu.VMEM((BATCH, D), jnp.float32),
            pltpu.VMEM((BATCH, D), jnp.float32),
            pltpu.VMEM((BS, DW), jnp.int32),
            pltpu.VMEM((BS, DW), jnp.int32),
            pltpu.VMEM((BS, DW), jnp.int32),
            pltpu.VMEM((BS, DW), jnp.int32),
            pltpu.VMEM((D,), jnp.float32),
            pltpu.SemaphoreType.DMA,
            pltpu.SemaphoreType.DMA,
            pltpu.SemaphoreType.DMA,
            pltpu.SemaphoreType.DMA,
            pltpu.SemaphoreType.DMA,
            pltpu.SemaphoreType.DMA,
        ],
    )
    def k(yb_hbm, adj_hbm, b_hbm, out_hbm,
          yb_sp, adj_v, my_v, ob0, ob1, r0, r1, r2, r3, b_v,
          s0, s1, s2, s3, os0, os1):
        bufs = (r0, r1, r2, r3)
        sems = (s0, s1, s2, s3)
        obufs = (ob0, ob1)
        osems = (os0, os1)
        sid = lax.axis_index("s")
        wid = sid * 2 + lax.axis_index("c")
        base = wid * CHUNK
        sl16 = pl.ds(sid * (N // 16), N // 16)
        pltpu.sync_copy(yb_hbm.at[sl16], yb_sp.at[sl16])
        for q in range(4):
            @pl.when(base + (q + 1) * QUARTER <= N)
            def _(q=q):
                src = pl.ds((base + q * QUARTER) * S, QUARTER * S)
                pltpu.sync_copy(adj_hbm.at[src],
                                adj_v.at[pl.ds(q * QUARTER * S, QUARTER * S)])

        pltpu.sync_copy(b_hbm, b_v)
        plsc.subcore_barrier()
        for q in range(4):
            @pl.when(base + (q + 1) * QUARTER <= N)
            def _(q=q):
                src = pl.ds(base + q * QUARTER, QUARTER)
                pltpu.sync_copy(yb_sp.at[src],
                                my_v.at[pl.ds(q * QUARTER, QUARTER)])

        def gather(s, buf, sem):
            idx = adj_v.at[pl.ds(s * BS, BS)]
            return pltpu.make_async_copy(yb_sp.at[idx], buf, sem)

        def ocopy(s, q):
            dst = out_hbm.at[pl.ds(base + s * BATCH, BATCH)]
            return pltpu.make_async_copy(obufs[q], dst, osems[q])

        def accum(s, buf, ob):
            for t in range(BATCH):
                i = s * BATCH + t
                row = t * S

                def rowloop(jj, accs, row=row, buf=buf):
                    accs = list(accs)
                    r = row + jj * 4
                    for u in range(4):
                        for g in range(NG):
                            v = buf[r + u, pl.ds(g * LANES, LANES)]
                            lo, hi = _unpack2(v)
                            accs[g] = accs[g] + lo
                            accs[g + NG] = accs[g + NG] + hi
                    return tuple(accs)

                zero = jnp.zeros((LANES,), jnp.float32)
                accs = lax.fori_loop(0, S // 4, rowloop, (zero,) * NCH)
                for g in range(NG):
                    v = my_v[i, pl.ds(g * LANES, LANES)]
                    lo, hi = _unpack2(v)
                    sl0 = pl.ds(g * LANES, LANES)
                    sl1 = pl.ds((g + NG) * LANES, LANES)
                    ob[t, sl0] = 0.5 * lo + (0.5 / S) * accs[g] + b_v[sl0]
                    ob[t, sl1] = 0.5 * hi + (0.5 / S) * accs[g + NG] + b_v[sl1]

        trips = jnp.where(base + CHUNK <= N, NSTEPS // NBUF,
                          NSTEPS // NBUF // 4)
        for p in range(NBUF):
            gather(p, bufs[p], sems[p]).start()

        def body(g4, carry):
            for p in range(NBUF):
                s = NBUF * g4 + p
                q = p % 2
                gather(s, bufs[p], sems[p]).wait()

                @pl.when(s >= 2)
                def _(s=s, q=q):
                    ocopy(s - 2, q).wait()

                accum(s, bufs[p], obufs[q])
                ocopy(s, q).start()

                @pl.when((s < NSTEPS - NBUF) & (g4 < trips - 1))
                def _(s=s, p=p):
                    gather(s + NBUF, bufs[p], sems[p]).start()

            return carry

        lax.fori_loop(0, trips, body, 0)
        laststep = trips * NBUF
        ocopy(laststep - 2, 0).wait()
        ocopy(laststep - 1, 1).wait()

    return k(ybi, adj_flat, b)


def kernel(x, neighbor_adj, W, b):
    ybi = _matmul_pack(x, W)
    adj_flat = neighbor_adj.astype(jnp.int32).reshape(N * S)
    return _sc_gather_combine(ybi, adj_flat, b)
